# baseline (speedup 1.0000x reference)
import functools

import jax
import jax.numpy as jnp
from jax import lax
from jax.experimental import pallas as pl
from jax.experimental.pallas import tpu as pltpu
from jax.experimental.pallas import tpu_sc as plsc

_NC = 2
_NS = 16
_NW = _NC * _NS
_CH = 32
_NB = 3

_TC_ROWS = 4096
_RB = 64
_NSEM = 8


def _sc_gather(idx, table, d):
    (n_rows,) = idx.shape
    b_per_w = n_rows // _NW
    n_chunks = b_per_w // _CH
    mesh = plsc.VectorSubcoreMesh(core_axis_name="c", subcore_axis_name="s")

    @functools.partial(
        pl.kernel,
        out_type=jax.ShapeDtypeStruct((n_rows, d), jnp.float32),
        mesh=mesh,
        scratch_types=[
            pltpu.VMEM((b_per_w,), jnp.int32),
            pltpu.VMEM((_NB, _CH, d), jnp.float32),
            pltpu.SemaphoreType.DMA((_NB,)),
            pltpu.SemaphoreType.DMA((_NB,)),
        ],
    )
    def k(idx_hbm, table_hbm, out_hbm, idx_v, bufs, gsem, osem):
        wid = lax.axis_index("s") * _NC + lax.axis_index("c")
        base = wid * b_per_w
        pltpu.sync_copy(idx_hbm.at[pl.ds(base, b_per_w)], idx_v)

        def gather(c, b):
            return pltpu.async_copy(
                table_hbm.at[idx_v.at[pl.ds(c * _CH, _CH)]], bufs.at[b], gsem.at[b]
            )

        def put(c, b):
            return pltpu.async_copy(
                bufs.at[b], out_hbm.at[pl.ds(base + c * _CH, _CH)], osem.at[b]
            )

        gdesc = [None] * _NB
        odesc = [None] * _NB
        for c in range(min(_NB - 1, n_chunks)):
            gdesc[c % _NB] = gather(c, c % _NB)
        for c in range(n_chunks):
            b = c % _NB
            nc = c + _NB - 1
            if nc < n_chunks:
                fb = nc % _NB
                if odesc[fb] is not None:
                    odesc[fb].wait()
                gdesc[fb] = gather(nc, fb)
            gdesc[b].wait()
            odesc[b] = put(c, b)
        for c in range(max(0, n_chunks - _NB), n_chunks):
            odesc[c % _NB].wait()

    return k(idx, table)


def _tc_gather(idx, table, d):
    (n_rows,) = idx.shape

    def body(ids_ref, table_ref, out_ref, sems):
        g = pl.program_id(0)
        descs = []
        for i in range(_RB):
            descs.append(
                pltpu.make_async_copy(
                    table_ref.at[pl.ds(ids_ref[g * _RB + i], 1), :],
                    out_ref.at[pl.ds(i, 1), :],
                    sems.at[i % _NSEM],
                )
            )
        for dsc in descs:
            dsc.start()
        for dsc in descs:
            dsc.wait()

    return pl.pallas_call(
        body,
        grid_spec=pltpu.PrefetchScalarGridSpec(
            num_scalar_prefetch=1,
            grid=(n_rows // _RB,),
            in_specs=[pl.BlockSpec(memory_space=pltpu.HBM)],
            out_specs=pl.BlockSpec((_RB, d), lambda g, ids: (g, 0)),
            scratch_shapes=[pltpu.SemaphoreType.DMA((_NSEM,))],
        ),
        out_shape=jax.ShapeDtypeStruct((n_rows, d), jnp.float32),
    )(idx, table)


@jax.jit
def _gather(input_ids, table):
    bsz, seq = input_ids.shape
    _, d = table.shape
    ids = input_ids.reshape(bsz * seq)
    tc_out = _tc_gather(ids[:_TC_ROWS], table, d)
    sc_out = _sc_gather(ids[_TC_ROWS:], table, d)
    return jnp.concatenate([tc_out, sc_out], axis=0).reshape(bsz, seq, d)


def kernel(input_ids, shared_weight):
    return _gather(input_ids, shared_weight)


# speedup vs baseline: 2.2904x; 2.2904x over previous
import functools

import jax
import jax.numpy as jnp
from jax import lax
from jax.experimental import pallas as pl
from jax.experimental.pallas import tpu as pltpu
from jax.experimental.pallas import tpu_sc as plsc

_NC = 2
_NS = 16
_NW = _NC * _NS
_CH = 16
_NB = 3


@jax.jit
def _sc_gather(idx, table):
    bsz, seq = idx.shape
    _, d = table.shape
    n_rows = bsz * seq
    b_per_w = n_rows // _NW
    w_per_b = seq // b_per_w
    n_chunks = b_per_w // _CH
    mesh = plsc.VectorSubcoreMesh(core_axis_name="c", subcore_axis_name="s")

    @functools.partial(
        pl.kernel,
        out_type=jax.ShapeDtypeStruct((bsz, seq, d), jnp.float32),
        mesh=mesh,
        scratch_types=[
            pltpu.VMEM((b_per_w,), jnp.int32),
            pltpu.VMEM((_NB, _CH, d), jnp.float32),
            pltpu.SemaphoreType.DMA((_NB,)),
            pltpu.SemaphoreType.DMA((_NB,)),
        ],
    )
    def k(idx_hbm, table_hbm, out_hbm, idx_v, bufs, gsem, osem):
        wid = lax.axis_index("s") * _NC + lax.axis_index("c")
        row = wid // w_per_b
        col = (wid % w_per_b) * b_per_w
        pltpu.sync_copy(idx_hbm.at[row, pl.ds(col, b_per_w)], idx_v)

        def gather(c, b):
            return pltpu.async_copy(
                table_hbm.at[idx_v.at[pl.ds(c * _CH, _CH)]], bufs.at[b], gsem.at[b]
            )

        def put(c, b):
            return pltpu.async_copy(
                bufs.at[b], out_hbm.at[row, pl.ds(col + c * _CH, _CH)], osem.at[b]
            )

        gdesc = [None] * _NB
        odesc = [None] * _NB
        for c in range(min(_NB - 1, n_chunks)):
            gdesc[c % _NB] = gather(c, c % _NB)
        for c in range(n_chunks):
            b = c % _NB
            nc = c + _NB - 1
            if nc < n_chunks:
                fb = nc % _NB
                if odesc[fb] is not None:
                    odesc[fb].wait()
                gdesc[fb] = gather(nc, fb)
            gdesc[b].wait()
            odesc[b] = put(c, b)
        for c in range(max(0, n_chunks - _NB), n_chunks):
            odesc[c % _NB].wait()

    return k(idx, table)


def kernel(input_ids, shared_weight):
    return _sc_gather(input_ids, shared_weight)
